# Initial kernel scaffold; baseline (speedup 1.0000x reference)
import functools

import jax
import jax.numpy as jnp
from jax import lax
from jax.experimental import pallas as pl
from jax.experimental.pallas import tpu as pltpu
from jax.experimental.pallas import tpu_sc as plsc

N_NODES = 10000
N_EDGES = 320000
D_IN = 128
D_HID = 16

NC = 2
NS = 16
NW = NC * NS
EPW = N_EDGES // NW
K = 80
NCHUNK = EPW // K
RPS = N_NODES // NS
DEG_PAD = 10240
DEG_PS = DEG_PAD // NS

_mesh = plsc.VectorSubcoreMesh(core_axis_name="c", subcore_axis_name="s")


@functools.partial(
    pl.kernel,
    out_type=jax.ShapeDtypeStruct((NC, NS, DEG_PS), jnp.float32),
    mesh=_mesh,
    scratch_types=[
        pltpu.VMEM((NCHUNK, K), jnp.int32),
        pltpu.VMEM((K,), jnp.float32),
        pltpu.VMEM((DEG_PS,), jnp.float32),
        pltpu.VMEM_SHARED((DEG_PAD,), jnp.float32),
    ],
)
def _deg_kernel(dst_hbm, out_hbm, idx_v, ones_v, zero_v, acc_sh):
    cid = lax.axis_index("c")
    sid = lax.axis_index("s")
    wid = sid * NC + cid

    @pl.loop(0, K, step=16)
    def _(i):
        ones_v.at[pl.ds(i, 16)][...] = jnp.ones((16,), jnp.float32)

    @pl.loop(0, DEG_PS, step=16)
    def _(i):
        zero_v.at[pl.ds(i, 16)][...] = jnp.zeros((16,), jnp.float32)

    pltpu.sync_copy(zero_v, acc_sh.at[pl.ds(sid * DEG_PS, DEG_PS)])
    pltpu.sync_copy(dst_hbm.at[wid], idx_v)
    plsc.subcore_barrier()

    @pl.loop(0, NCHUNK)
    def _(j):
        pltpu.sync_copy(ones_v, acc_sh.at[idx_v.at[j]], add=True)

    plsc.subcore_barrier()
    pltpu.sync_copy(acc_sh.at[pl.ds(sid * DEG_PS, DEG_PS)], out_hbm.at[cid, sid])


@functools.partial(
    pl.kernel,
    out_type=jax.ShapeDtypeStruct((NC, NS, RPS, D_HID), jnp.float32),
    mesh=_mesh,
    scratch_types=[
        pltpu.VMEM((NCHUNK, K), jnp.int32),
        pltpu.VMEM((NCHUNK, K), jnp.int32),
        pltpu.VMEM((K, D_HID), jnp.float32),
        pltpu.VMEM((RPS, D_HID), jnp.float32),
        pltpu.VMEM_SHARED((N_NODES, D_HID), jnp.float32),
    ],
)
def _agg_kernel(table_hbm, src_hbm, dst_hbm, out_hbm,
                src_v, dst_v, rows_v, zrows_v, acc_sh):
    cid = lax.axis_index("c")
    sid = lax.axis_index("s")
    wid = sid * NC + cid

    @pl.loop(0, RPS)
    def _(r):
        zrows_v.at[r][...] = jnp.zeros((16,), jnp.float32)

    pltpu.sync_copy(zrows_v, acc_sh.at[pl.ds(sid * RPS, RPS)])
    pltpu.sync_copy(src_hbm.at[wid], src_v)
    pltpu.sync_copy(dst_hbm.at[wid], dst_v)
    plsc.subcore_barrier()

    @pl.loop(0, NCHUNK)
    def _(j):
        pltpu.sync_copy(table_hbm.at[src_v.at[j]], rows_v)
        pltpu.sync_copy(rows_v, acc_sh.at[dst_v.at[j]], add=True)

    plsc.subcore_barrier()
    pltpu.sync_copy(acc_sh.at[pl.ds(sid * RPS, RPS)], out_hbm.at[cid, sid])


def _mm_body(x_ref, w_ref, o_ref):
    o_ref[...] = lax.dot_general(
        x_ref[...], w_ref[...], (((1,), (0,)), ((), ())),
        precision=lax.Precision.HIGHEST, preferred_element_type=jnp.float32)


def _scale1_body(deg_ref, xw_ref, dinv_ref, yw_ref):
    d = deg_ref[0] + deg_ref[1] + 1.0
    dinv = lax.rsqrt(d)
    dinv_ref[...] = dinv
    yw_ref[...] = xw_ref[...] * dinv


def _mid_body(agg_ref, yw1_ref, dinv_ref, b1_ref, w2_ref, yw2_ref):
    t = agg_ref[0] + agg_ref[1] + yw1_ref[...]
    h = jnp.maximum(t * dinv_ref[...] + b1_ref[...], 0.0)
    hw = lax.dot_general(
        h, w2_ref[...], (((1,), (0,)), ((), ())),
        precision=lax.Precision.HIGHEST, preferred_element_type=jnp.float32)
    yw2_ref[...] = hw * dinv_ref[...]


def _final_body(agg_ref, yw2_ref, dinv_ref, b2_ref, out_ref):
    t = agg_ref[0] + agg_ref[1] + yw2_ref[...]
    out_ref[...] = t * dinv_ref[...] + b2_ref[...]


def kernel(x, edge_index, W1, b1, W2, b2):
    f32 = jnp.float32
    src = edge_index[0].astype(jnp.int32).reshape(NW, NCHUNK, K)
    dst = edge_index[1].astype(jnp.int32).reshape(NW, NCHUNK, K)

    deg_parts = _deg_kernel(dst)
    deg_col = deg_parts.reshape(NC, DEG_PAD)[:, :N_NODES].reshape(
        NC, N_NODES, 1)

    xw1 = pl.pallas_call(
        _mm_body,
        out_shape=jax.ShapeDtypeStruct((N_NODES, D_HID), f32),
    )(x, W1)

    dinv_col, yw1 = pl.pallas_call(
        _scale1_body,
        out_shape=(jax.ShapeDtypeStruct((N_NODES, 1), f32),
                   jax.ShapeDtypeStruct((N_NODES, D_HID), f32)),
    )(deg_col, xw1)

    agg1 = _agg_kernel(yw1, src, dst).reshape(NC, N_NODES, D_HID)

    yw2 = pl.pallas_call(
        _mid_body,
        out_shape=jax.ShapeDtypeStruct((N_NODES, D_HID), f32),
    )(agg1, yw1, dinv_col, b1.reshape(1, D_HID), W2)

    agg2 = _agg_kernel(yw2, src, dst).reshape(NC, N_NODES, D_HID)

    out = pl.pallas_call(
        _final_body,
        out_shape=jax.ShapeDtypeStruct((N_NODES, D_HID), f32),
    )(agg2, yw2, dinv_col, b2.reshape(1, D_HID))
    return out


# speedup vs baseline: 29.1736x; 29.1736x over previous
import functools

import jax
import jax.numpy as jnp
from jax import lax
from jax.experimental import pallas as pl
from jax.experimental.pallas import tpu as pltpu
from jax.experimental.pallas import tpu_sc as plsc

N_NODES = 10000
N_EDGES = 320000
D_IN = 128
D_HID = 16

NC = 2
NS = 16
NW = NC * NS
EPW = N_EDGES // NW
K = 80
NCHUNK = EPW // K
RPS = N_NODES // NS
DEG_PAD = 10240
DEG_PS = DEG_PAD // NS

_mesh = plsc.VectorSubcoreMesh(core_axis_name="c", subcore_axis_name="s")
_sc_params = pltpu.CompilerParams(use_tc_tiling_on_sc=False)


@functools.partial(
    pl.kernel,
    out_type=jax.ShapeDtypeStruct((NC, NS, DEG_PS), jnp.float32),
    mesh=_mesh,
    scratch_types=[
        pltpu.VMEM((NCHUNK, K), jnp.int32),
        pltpu.VMEM((K,), jnp.float32),
        pltpu.VMEM((DEG_PS,), jnp.float32),
        pltpu.VMEM_SHARED((DEG_PAD,), jnp.float32),
    ],
    compiler_params=_sc_params,
)
def _deg_kernel(dst_hbm, out_hbm, idx_v, ones_v, zero_v, acc_sh):
    cid = lax.axis_index("c")
    sid = lax.axis_index("s")
    wid = sid * NC + cid

    @pl.loop(0, K, step=16)
    def _(i):
        ones_v.at[pl.ds(i, 16)][...] = jnp.ones((16,), jnp.float32)

    @pl.loop(0, DEG_PS, step=16)
    def _(i):
        zero_v.at[pl.ds(i, 16)][...] = jnp.zeros((16,), jnp.float32)

    pltpu.sync_copy(zero_v, acc_sh.at[pl.ds(sid * DEG_PS, DEG_PS)])
    pltpu.sync_copy(dst_hbm.at[wid], idx_v)
    plsc.subcore_barrier()

    @pl.loop(0, NCHUNK)
    def _(j):
        pltpu.sync_copy(ones_v, acc_sh.at[idx_v.at[j]], add=True)

    plsc.subcore_barrier()
    pltpu.sync_copy(acc_sh.at[pl.ds(sid * DEG_PS, DEG_PS)], out_hbm.at[cid, sid])


@functools.partial(
    pl.kernel,
    out_type=jax.ShapeDtypeStruct((NC, NS, RPS, D_HID), jnp.float32),
    mesh=_mesh,
    scratch_types=[
        pltpu.VMEM((NCHUNK, K), jnp.int32),
        pltpu.VMEM((NCHUNK, K), jnp.int32),
        pltpu.VMEM((K, D_HID), jnp.float32),
        pltpu.VMEM((RPS, D_HID), jnp.float32),
        pltpu.VMEM_SHARED((N_NODES, D_HID), jnp.float32),
    ],
    compiler_params=_sc_params,
)
def _agg_kernel(table_hbm, src_hbm, dst_hbm, out_hbm,
                src_v, dst_v, rows_v, zrows_v, acc_sh):
    cid = lax.axis_index("c")
    sid = lax.axis_index("s")
    wid = sid * NC + cid

    @pl.loop(0, RPS)
    def _(r):
        zrows_v.at[r][...] = jnp.zeros((16,), jnp.float32)

    pltpu.sync_copy(zrows_v, acc_sh.at[pl.ds(sid * RPS, RPS)])
    pltpu.sync_copy(src_hbm.at[wid], src_v)
    pltpu.sync_copy(dst_hbm.at[wid], dst_v)
    plsc.subcore_barrier()

    @pl.loop(0, NCHUNK)
    def _(j):
        pltpu.sync_copy(table_hbm.at[src_v.at[j]], rows_v)
        pltpu.sync_copy(rows_v, acc_sh.at[dst_v.at[j]], add=True)

    plsc.subcore_barrier()
    pltpu.sync_copy(acc_sh.at[pl.ds(sid * RPS, RPS)], out_hbm.at[cid, sid])


def _mm_body(x_ref, w_ref, o_ref):
    o_ref[...] = lax.dot_general(
        x_ref[...], w_ref[...], (((1,), (0,)), ((), ())),
        precision=lax.Precision.HIGHEST, preferred_element_type=jnp.float32)


def _scale1_body(deg_ref, xw_ref, dinv_ref, yw_ref):
    d = deg_ref[0] + deg_ref[1] + 1.0
    dinv = lax.rsqrt(d)
    dinv_ref[...] = dinv
    yw_ref[...] = xw_ref[...] * dinv


def _mid_body(agg_ref, yw1_ref, dinv_ref, b1_ref, w2_ref, yw2_ref):
    t = agg_ref[0] + agg_ref[1] + yw1_ref[...]
    h = jnp.maximum(t * dinv_ref[...] + b1_ref[...], 0.0)
    hw = lax.dot_general(
        h, w2_ref[...], (((1,), (0,)), ((), ())),
        precision=lax.Precision.HIGHEST, preferred_element_type=jnp.float32)
    yw2_ref[...] = hw * dinv_ref[...]


def _final_body(agg_ref, yw2_ref, dinv_ref, b2_ref, out_ref):
    t = agg_ref[0] + agg_ref[1] + yw2_ref[...]
    out_ref[...] = t * dinv_ref[...] + b2_ref[...]


def kernel(x, edge_index, W1, b1, W2, b2):
    f32 = jnp.float32
    src = edge_index[0].astype(jnp.int32).reshape(NW, NCHUNK, K)
    dst = edge_index[1].astype(jnp.int32).reshape(NW, NCHUNK, K)

    deg_parts = _deg_kernel(dst)
    deg_col = deg_parts.reshape(NC, DEG_PAD)[:, :N_NODES].reshape(
        NC, N_NODES, 1)

    xw1 = pl.pallas_call(
        _mm_body,
        out_shape=jax.ShapeDtypeStruct((N_NODES, D_HID), f32),
    )(x, W1)

    dinv_col, yw1 = pl.pallas_call(
        _scale1_body,
        out_shape=(jax.ShapeDtypeStruct((N_NODES, 1), f32),
                   jax.ShapeDtypeStruct((N_NODES, D_HID), f32)),
    )(deg_col, xw1)

    agg1 = _agg_kernel(yw1, src, dst).reshape(NC, N_NODES, D_HID)

    yw2 = pl.pallas_call(
        _mid_body,
        out_shape=jax.ShapeDtypeStruct((N_NODES, D_HID), f32),
    )(agg1, yw1, dinv_col, b1.reshape(1, D_HID), W2)

    agg2 = _agg_kernel(yw2, src, dst).reshape(NC, N_NODES, D_HID)

    out = pl.pallas_call(
        _final_body,
        out_shape=jax.ShapeDtypeStruct((N_NODES, D_HID), f32),
    )(agg2, yw2, dinv_col, b2.reshape(1, D_HID))
    return out


# speedup vs baseline: 52.6463x; 1.8046x over previous
import functools

import jax
import jax.numpy as jnp
from jax import lax
from jax.experimental import pallas as pl
from jax.experimental.pallas import tpu as pltpu
from jax.experimental.pallas import tpu_sc as plsc

N_NODES = 10000
N_EDGES = 320000
D_IN = 128
D_HID = 16

NC = 2
NS = 16
NW = NC * NS
EPW = N_EDGES // NW
K = 80
NCHUNK = EPW // K
RPS = N_NODES // NS
DEG_PAD = 10240
DEG_PS = DEG_PAD // NS

_mesh = plsc.VectorSubcoreMesh(core_axis_name="c", subcore_axis_name="s")
_sc_params = pltpu.CompilerParams(use_tc_tiling_on_sc=False)


@functools.partial(
    pl.kernel,
    out_type=jax.ShapeDtypeStruct((NC, NS, DEG_PS), jnp.float32),
    mesh=_mesh,
    scratch_types=[
        pltpu.VMEM((NCHUNK, K), jnp.int32),
        pltpu.VMEM((K,), jnp.float32),
        pltpu.VMEM((DEG_PS,), jnp.float32),
        pltpu.VMEM_SHARED((DEG_PAD,), jnp.float32),
    ],
    compiler_params=_sc_params,
)
def _deg_kernel(dst_hbm, out_hbm, idx_v, ones_v, zero_v, acc_sh):
    cid = lax.axis_index("c")
    sid = lax.axis_index("s")
    wid = sid * NC + cid

    @pl.loop(0, K, step=16)
    def _(i):
        ones_v.at[pl.ds(i, 16)][...] = jnp.ones((16,), jnp.float32)

    @pl.loop(0, DEG_PS, step=16)
    def _(i):
        zero_v.at[pl.ds(i, 16)][...] = jnp.zeros((16,), jnp.float32)

    pltpu.sync_copy(zero_v, acc_sh.at[pl.ds(sid * DEG_PS, DEG_PS)])
    pltpu.sync_copy(dst_hbm.at[wid], idx_v)
    plsc.subcore_barrier()

    @pl.loop(0, NCHUNK)
    def _(j):
        pltpu.sync_copy(ones_v, acc_sh.at[idx_v.at[j]], add=True)

    plsc.subcore_barrier()
    pltpu.sync_copy(acc_sh.at[pl.ds(sid * DEG_PS, DEG_PS)], out_hbm.at[cid, sid])


RING = 5


@functools.partial(
    pl.kernel,
    out_type=jax.ShapeDtypeStruct((NC, NS, RPS, D_HID), jnp.float32),
    mesh=_mesh,
    scratch_types=[
        pltpu.VMEM((NCHUNK, K), jnp.int32),
        pltpu.VMEM((NCHUNK, K), jnp.int32),
    ] + [pltpu.VMEM((K, D_HID), jnp.float32) for _ in range(RING)] + [
        pltpu.VMEM((RPS, D_HID), jnp.float32),
        pltpu.VMEM_SHARED((N_NODES, D_HID), jnp.float32),
        pltpu.SemaphoreType.DMA((RING,)),
        pltpu.SemaphoreType.DMA((RING,)),
    ],
    compiler_params=_sc_params,
)
def _agg_kernel(table_hbm, src_hbm, dst_hbm, out_hbm,
                src_v, dst_v, r0, r1, r2, r3, r4, zrows_v, acc_sh,
                gsem, ssem):
    rows = (r0, r1, r2, r3, r4)
    cid = lax.axis_index("c")
    sid = lax.axis_index("s")
    wid = sid * NC + cid

    @pl.loop(0, RPS)
    def _(r):
        zrows_v.at[r][...] = jnp.zeros((16,), jnp.float32)

    pltpu.sync_copy(zrows_v, acc_sh.at[pl.ds(sid * RPS, RPS)])
    pltpu.sync_copy(src_hbm.at[wid], src_v)
    pltpu.sync_copy(dst_hbm.at[wid], dst_v)
    plsc.subcore_barrier()

    def gather_start(c, b):
        pltpu.async_copy(table_hbm.at[src_v.at[c]], rows[b], gsem.at[b])

    def gather_wait(c, b):
        pltpu.make_async_copy(table_hbm.at[src_v.at[c]], rows[b],
                              gsem.at[b]).wait()

    def scat_start(c, b):
        pltpu.async_copy(rows[b], acc_sh.at[dst_v.at[c]], ssem.at[b],
                         add=True)

    def scat_wait(c, b):
        pltpu.make_async_copy(rows[b], acc_sh.at[dst_v.at[c]],
                              ssem.at[b]).wait()

    for b in range(RING):
        gather_start(b, b)

    @pl.loop(0, NCHUNK - RING, step=RING)
    def _(j):
        for b in range(RING):
            gather_wait(j + b, b)
            scat_start(j + b, b)
        for b in range(RING):
            scat_wait(j + b, b)
            gather_start(j + RING + b, b)

    for b in range(RING):
        gather_wait(NCHUNK - RING + b, b)
        scat_start(NCHUNK - RING + b, b)
    for b in range(RING):
        scat_wait(NCHUNK - RING + b, b)

    plsc.subcore_barrier()
    pltpu.sync_copy(acc_sh.at[pl.ds(sid * RPS, RPS)], out_hbm.at[cid, sid])


def _mm_body(x_ref, w_ref, o_ref):
    o_ref[...] = lax.dot_general(
        x_ref[...], w_ref[...], (((1,), (0,)), ((), ())),
        precision=lax.Precision.HIGHEST, preferred_element_type=jnp.float32)


def _scale1_body(deg_ref, xw_ref, dinv_ref, yw_ref):
    d = deg_ref[0] + deg_ref[1] + 1.0
    dinv = lax.rsqrt(d)
    dinv_ref[...] = dinv
    yw_ref[...] = xw_ref[...] * dinv


def _mid_body(agg_ref, yw1_ref, dinv_ref, b1_ref, w2_ref, yw2_ref):
    t = agg_ref[0] + agg_ref[1] + yw1_ref[...]
    h = jnp.maximum(t * dinv_ref[...] + b1_ref[...], 0.0)
    hw = lax.dot_general(
        h, w2_ref[...], (((1,), (0,)), ((), ())),
        precision=lax.Precision.HIGHEST, preferred_element_type=jnp.float32)
    yw2_ref[...] = hw * dinv_ref[...]


def _final_body(agg_ref, yw2_ref, dinv_ref, b2_ref, out_ref):
    t = agg_ref[0] + agg_ref[1] + yw2_ref[...]
    out_ref[...] = t * dinv_ref[...] + b2_ref[...]


def kernel(x, edge_index, W1, b1, W2, b2):
    f32 = jnp.float32
    src = edge_index[0].astype(jnp.int32).reshape(NW, NCHUNK, K)
    dst = edge_index[1].astype(jnp.int32).reshape(NW, NCHUNK, K)

    deg_parts = _deg_kernel(dst)
    deg_col = deg_parts.reshape(NC, DEG_PAD)[:, :N_NODES].reshape(
        NC, N_NODES, 1)

    xw1 = pl.pallas_call(
        _mm_body,
        out_shape=jax.ShapeDtypeStruct((N_NODES, D_HID), f32),
    )(x, W1)

    dinv_col, yw1 = pl.pallas_call(
        _scale1_body,
        out_shape=(jax.ShapeDtypeStruct((N_NODES, 1), f32),
                   jax.ShapeDtypeStruct((N_NODES, D_HID), f32)),
    )(deg_col, xw1)

    agg1 = _agg_kernel(yw1, src, dst).reshape(NC, N_NODES, D_HID)

    yw2 = pl.pallas_call(
        _mid_body,
        out_shape=jax.ShapeDtypeStruct((N_NODES, D_HID), f32),
    )(agg1, yw1, dinv_col, b1.reshape(1, D_HID), W2)

    agg2 = _agg_kernel(yw2, src, dst).reshape(NC, N_NODES, D_HID)

    out = pl.pallas_call(
        _final_body,
        out_shape=jax.ShapeDtypeStruct((N_NODES, D_HID), f32),
    )(agg2, yw2, dinv_col, b2.reshape(1, D_HID))
    return out
